# Initial kernel scaffold; baseline (speedup 1.0000x reference)
import jax
import jax.numpy as jnp
from jax.experimental import pallas as pl


def kernel(feat, segment_ids):
    raise NotImplementedError("write your pallas kernel here")


# speedup vs baseline: 4.7084x; 4.7084x over previous
import functools

import jax
import jax.numpy as jnp
from jax import lax
from jax.experimental import pallas as pl
from jax.experimental.pallas import tpu as pltpu
from jax.experimental.pallas import tpu_sc as plsc

NUM_SEGMENTS = 128
D_FEAT = 128
N_ROWS = 100000
WIN = 80
NUM_WINDOWS = N_ROWS // WIN
NC = 2
NS = 16
NW = NC * NS
ITERS = (NUM_WINDOWS + NW - 1) // NW


def _sc_segment_sums(feat, ids):
    mesh = plsc.VectorSubcoreMesh(core_axis_name="c", subcore_axis_name="s")

    @functools.partial(
        pl.kernel,
        out_type=(
            jax.ShapeDtypeStruct((NC, NUM_SEGMENTS, D_FEAT), jnp.float32),
            jax.ShapeDtypeStruct((NC, NUM_SEGMENTS), jnp.float32),
        ),
        mesh=mesh,
        scratch_types=[
            pltpu.VMEM((WIN, D_FEAT), jnp.float32),
            pltpu.VMEM((WIN,), jnp.int32),
            pltpu.VMEM((WIN,), jnp.float32),
            pltpu.VMEM((NUM_SEGMENTS, D_FEAT), jnp.float32),
            pltpu.VMEM_SHARED((NUM_SEGMENTS, D_FEAT), jnp.float32),
            pltpu.VMEM_SHARED((NUM_SEGMENTS,), jnp.float32),
        ],
    )
    def seg_sum(feat_hbm, ids_hbm, out_sum, out_cnt,
                fbuf, idx_buf, ones_buf, zbuf, acc_sh, cnt_sh):
        c = lax.axis_index("c")
        s = lax.axis_index("s")
        w = s * NC + c

        ones16 = jnp.ones((16,), jnp.float32)
        for j in range(WIN // 16):
            ones_buf[pl.ds(j * 16, 16)] = ones16

        @pl.when(s == 0)
        def _():
            z16 = jnp.zeros((16,), jnp.float32)

            def zrow(i, carry):
                for j in range(D_FEAT // 16):
                    zbuf[i, pl.ds(j * 16, 16)] = z16
                return carry

            lax.fori_loop(0, NUM_SEGMENTS, zrow, 0)
            pltpu.sync_copy(zbuf, acc_sh)
            pltpu.sync_copy(zbuf.at[0], cnt_sh)

        plsc.subcore_barrier()

        def body(i, carry):
            t = w + i * NW

            @pl.when(t < NUM_WINDOWS)
            def _():
                base = t * WIN
                pltpu.sync_copy(feat_hbm.at[pl.ds(base, WIN)], fbuf)
                pltpu.sync_copy(ids_hbm.at[pl.ds(base, WIN)], idx_buf)
                pltpu.sync_copy(fbuf, acc_sh.at[idx_buf], add=True)
                pltpu.sync_copy(ones_buf, cnt_sh.at[idx_buf], add=True)

            return carry

        lax.fori_loop(0, ITERS, body, 0)

        plsc.subcore_barrier()

        @pl.when(s == 0)
        def _():
            pltpu.sync_copy(acc_sh, out_sum.at[c])
            pltpu.sync_copy(cnt_sh, out_cnt.at[c])

    return seg_sum(feat, ids)


def _merge_and_divide(sums, cnts):
    def combine(sum_ref, cnt_ref, out_ref):
        total = sum_ref[0] + sum_ref[1]
        cnt = cnt_ref[0] + cnt_ref[1]
        denom = jnp.maximum(cnt, 1.0)[:, None]
        out_ref[...] = total / denom

    return pl.pallas_call(
        combine,
        out_shape=jax.ShapeDtypeStruct((NUM_SEGMENTS, D_FEAT), jnp.float32),
    )(sums, cnts)


@jax.jit
def kernel(feat, segment_ids):
    ids = segment_ids.astype(jnp.int32)
    sums, cnts = _sc_segment_sums(feat, ids)
    return _merge_and_divide(sums, cnts)


# speedup vs baseline: 6.7977x; 1.4438x over previous
import functools

import jax
import jax.numpy as jnp
from jax import lax
from jax.experimental import pallas as pl
from jax.experimental.pallas import tpu as pltpu
from jax.experimental.pallas import tpu_sc as plsc

NUM_SEGMENTS = 128
D_FEAT = 128
N_ROWS = 100000
WIN = 80
NUM_WINDOWS = N_ROWS // WIN
NC = 2
NS = 16
NW = NC * NS
SLOTS = (NUM_WINDOWS + NW - 1) // NW
ROWS_PER_TILE = NUM_SEGMENTS // NS


def _sc_segment_sums(feat, ids2d):
    mesh = plsc.VectorSubcoreMesh(core_axis_name="c", subcore_axis_name="s")

    @functools.partial(
        pl.kernel,
        out_type=(
            jax.ShapeDtypeStruct((NC, NUM_SEGMENTS, D_FEAT), jnp.float32),
            jax.ShapeDtypeStruct((NW, NUM_SEGMENTS), jnp.float32),
        ),
        mesh=mesh,
        compiler_params=pltpu.CompilerParams(
            use_tc_tiling_on_sc=False, needs_layout_passes=False),
        scratch_types=[
            pltpu.VMEM((WIN, D_FEAT), jnp.float32),
            pltpu.VMEM((WIN, D_FEAT), jnp.float32),
            pltpu.VMEM((SLOTS, WIN), jnp.int32),
            pltpu.VMEM((NUM_SEGMENTS,), jnp.float32),
            pltpu.VMEM_SHARED((NUM_SEGMENTS, D_FEAT), jnp.float32),
            pltpu.SemaphoreType.DMA,
            pltpu.SemaphoreType.DMA,
        ],
    )
    def seg_sum(feat_hbm, ids_hbm, out_sum, out_cnt,
                fbuf_a, fbuf_b, idx_all, cnt_buf, acc_sh, sem_a, sem_b):
        c = lax.axis_index("c")
        s = lax.axis_index("s")
        w = s * NC + c

        n_w = jnp.where(w < 2, SLOTS, SLOTS - 1)
        b_w = (SLOTS - 1) * w + jnp.minimum(w, 2)
        pb = jnp.minimum(b_w, NUM_WINDOWS - SLOTS)
        shift = b_w - pb

        pltpu.sync_copy(ids_hbm.at[pl.ds(pb, SLOTS)], idx_all)

        z16 = jnp.zeros((16,), jnp.float32)
        for j in range(NUM_SEGMENTS // 16):
            cnt_buf[pl.ds(j * 16, 16)] = z16
        for i in range(ROWS_PER_TILE):
            for j in range(D_FEAT // 16):
                fbuf_a[i, pl.ds(j * 16, 16)] = z16
        pltpu.sync_copy(fbuf_a.at[pl.ds(0, ROWS_PER_TILE)],
                        acc_sh.at[pl.ds(s * ROWS_PER_TILE, ROWS_PER_TILE)])
        plsc.subcore_barrier()

        def win_base(l):
            return jnp.minimum(b_w + l, NUM_WINDOWS - 1) * WIN

        def gather(l, buf, sem):
            pltpu.make_async_copy(
                feat_hbm.at[pl.ds(win_base(l), WIN)], buf, sem).start()

        def wait(l, buf, sem):
            pltpu.make_async_copy(
                feat_hbm.at[pl.ds(win_base(l), WIN)], buf, sem).wait()

        ones16 = jnp.ones((16,), jnp.float32)

        def process(l, buf, sem):
            wait(l, buf, sem)

            @pl.when(l < n_w)
            def _():
                idx_row = idx_all.at[shift + l]
                for k in range(WIN // 16):
                    ids16 = idx_all[shift + l, pl.ds(k * 16, 16)]
                    plsc.addupdate_scatter(cnt_buf, [ids16], ones16)
                pltpu.sync_copy(buf, acc_sh.at[idx_row], add=True)

        gather(0, fbuf_a, sem_a)

        def body(i, carry):
            l0 = 2 * i
            l1 = 2 * i + 1

            @pl.when(l1 < SLOTS)
            def _():
                gather(l1, fbuf_b, sem_b)

            process(l0, fbuf_a, sem_a)

            @pl.when(l0 + 2 < SLOTS)
            def _():
                gather(l0 + 2, fbuf_a, sem_a)

            @pl.when(l1 < SLOTS)
            def _():
                process(l1, fbuf_b, sem_b)

            return carry

        lax.fori_loop(0, (SLOTS + 1) // 2, body, 0)

        pltpu.sync_copy(cnt_buf, out_cnt.at[w])
        plsc.subcore_barrier()

        @pl.when(s == 0)
        def _():
            pltpu.sync_copy(acc_sh, out_sum.at[c])

    return seg_sum(feat, ids2d)


def _merge_and_divide(sums, cnts):
    def combine(sum_ref, cnt_ref, out_ref):
        total = sum_ref[0] + sum_ref[1]
        cnt = jnp.sum(cnt_ref[...], axis=0)
        denom = jnp.maximum(cnt, 1.0)[:, None]
        out_ref[...] = total / denom

    return pl.pallas_call(
        combine,
        out_shape=jax.ShapeDtypeStruct((NUM_SEGMENTS, D_FEAT), jnp.float32),
    )(sums, cnts)


@jax.jit
def kernel(feat, segment_ids):
    ids2d = segment_ids.astype(jnp.int32).reshape(NUM_WINDOWS, WIN)
    sums, cnts = _sc_segment_sums(feat, ids2d)
    return _merge_and_divide(sums, cnts)
